# baseline (speedup 1.0000x reference)
import functools

import jax
import jax.numpy as jnp
from jax import lax
from jax.experimental import pallas as pl
from jax.experimental.pallas import tpu as pltpu
from jax.experimental.pallas import tpu_sc as plsc

VOCAB = 1000000
D = 32
B = 4096
L_SEQ = 50
LP = 56
NC = 2
NS = 16
NW = NC * NS
BPW = B // NW
IDX_W = BPW * LP


def _nn_body(xp_hbm, emb_hbm, wb_hbm, out_hbm,
             idx_v, rows0, rows1, wb_v, out_v, sem0, sem1):
    wid = lax.axis_index("s") * NC + lax.axis_index("c")
    ibase = wid * IDX_W

    pltpu.sync_copy(xp_hbm.at[pl.ds(ibase, IDX_W)], idx_v.at[pl.ds(0, IDX_W)])
    pltpu.sync_copy(wb_hbm, wb_v)

    zero16 = jnp.zeros((16,), jnp.int32)
    for k in range(4):
        idx_v[pl.ds(IDX_W + k * 16, 16)] = zero16

    w0 = wb_v[pl.ds(0, 16)]
    w1 = wb_v[pl.ds(16, 16)]
    bias = wb_v[pl.ds(32, 16)][0]
    inv_l = jnp.float32(1.0 / L_SEQ)

    def gather(b, buf, sem):
        return pltpu.make_async_copy(
            emb_hbm.at[idx_v.at[pl.ds(b * LP, L_SEQ)]], buf, sem)

    lanes = lax.iota(jnp.int32, 16)
    gather_dnums = lax.GatherDimensionNumbers(
        offset_dims=(), collapsed_slice_dims=(0,), start_index_map=(0,))

    def lane_shuffle(v, k):
        perm = lax.bitwise_xor(lanes, k)
        return lax.gather(v, perm[:, None], gather_dnums, slice_sizes=(1,),
                          mode=lax.GatherScatterMode.PROMISE_IN_BOUNDS)

    def reduce_one(buf):
        acc0 = buf[0, pl.ds(0, 16)]
        acc1 = buf[0, pl.ds(16, 16)]
        for l in range(1, L_SEQ):
            acc0 = acc0 + buf[l, pl.ds(0, 16)]
            acc1 = acc1 + buf[l, pl.ds(16, 16)]
        p = acc0 * w0 + acc1 * w1
        for k in (8, 4, 2, 1):
            p = p + lane_shuffle(p, k)
        return p * inv_l + bias

    gather(0, rows0, sem0).start()

    def loop_body(i, vec):
        b = 2 * i
        gather(b + 1, rows1, sem1).start()
        gather(b, rows0, sem0).wait()
        s0 = reduce_one(rows0)
        gather(b + 2, rows0, sem0).start()
        gather(b + 1, rows1, sem1).wait()
        s1 = reduce_one(rows1)
        j = lax.rem(i, 8)
        vec = jnp.where(lanes == 2 * j, s0, vec)
        vec = jnp.where(lanes == 2 * j + 1, s1, vec)

        @pl.when(j == 7)
        def _store():
            out_v[pl.ds((i // 8) * 16, 16)] = vec

        return vec

    lax.fori_loop(0, BPW // 2, loop_body, jnp.zeros((16,), jnp.float32))

    gather(BPW, rows0, sem0).wait()

    pltpu.sync_copy(out_v, out_hbm.at[pl.ds(wid * BPW, BPW)])


_mesh = plsc.VectorSubcoreMesh(
    core_axis_name="c", subcore_axis_name="s", num_cores=NC, num_subcores=NS)

_nn_kernel = functools.partial(
    pl.kernel,
    out_type=jax.ShapeDtypeStruct((B,), jnp.float32),
    mesh=_mesh,
    compiler_params=pltpu.CompilerParams(use_tc_tiling_on_sc=False),
    scratch_types=[
        pltpu.VMEM((IDX_W + 64,), jnp.int32),
        pltpu.VMEM((L_SEQ, D), jnp.float32),
        pltpu.VMEM((L_SEQ, D), jnp.float32),
        pltpu.VMEM((48,), jnp.float32),
        pltpu.VMEM((BPW,), jnp.float32),
        pltpu.SemaphoreType.DMA,
        pltpu.SemaphoreType.DMA,
    ],
)(_nn_body)


@jax.jit
def kernel(x, emb, W, b):
    xp = jnp.pad(x.astype(jnp.int32), ((0, 0), (0, LP - L_SEQ)))
    x_flat = xp.reshape(B * LP)
    wb = jnp.concatenate([W.reshape(D), jnp.broadcast_to(b, (16,))])
    out = _nn_kernel(x_flat, emb, wb)
    return out.reshape(B, 1)


# speedup vs baseline: 1.0235x; 1.0235x over previous
import functools

import jax
import jax.numpy as jnp
from jax import lax
from jax.experimental import pallas as pl
from jax.experimental.pallas import tpu as pltpu
from jax.experimental.pallas import tpu_sc as plsc

VOCAB = 1000000
D = 32
B = 4096
L_SEQ = 50
NC = 2
NS = 16
NW = NC * NS
BPW = B // NW


def _nn_body(x_hbm, emb_hbm, wb_hbm, out_hbm,
             idx_v, rows0, rows1, wb_v, out_v, sem0, sem1):
    wid = lax.axis_index("s") * NC + lax.axis_index("c")

    pltpu.sync_copy(x_hbm.at[pl.ds(wid * BPW, BPW)], idx_v)
    pltpu.sync_copy(wb_hbm, wb_v)

    w0 = wb_v[pl.ds(0, 16)]
    w1 = wb_v[pl.ds(16, 16)]
    bias = wb_v[pl.ds(32, 16)][0]
    inv_l = jnp.float32(1.0 / L_SEQ)

    lanes = lax.iota(jnp.int32, 16)
    gather_dnums = lax.GatherDimensionNumbers(
        offset_dims=(), collapsed_slice_dims=(0,), start_index_map=(0,))

    def lane_shuffle(v, k):
        perm = lax.bitwise_xor(lanes, k)
        return lax.gather(v, perm[:, None], gather_dnums, slice_sizes=(1,),
                          mode=lax.GatherScatterMode.PROMISE_IN_BOUNDS)

    def gather(b, buf, sem):
        return pltpu.make_async_copy(emb_hbm.at[idx_v.at[b]], buf, sem)

    def reduce_one(buf):
        acc0 = buf[0, pl.ds(0, 16)]
        acc1 = buf[0, pl.ds(16, 16)]
        for l in range(1, L_SEQ):
            acc0 = acc0 + buf[l, pl.ds(0, 16)]
            acc1 = acc1 + buf[l, pl.ds(16, 16)]
        p = acc0 * w0 + acc1 * w1
        for k in (8, 4, 2, 1):
            p = p + lane_shuffle(p, k)
        return p * inv_l + bias

    def pack(vec, j, s):
        return jnp.where(lanes == j, s, vec)

    gather(0, rows0, sem0).start()

    def loop_body(i, vec):
        b = 2 * i
        gather(b + 1, rows1, sem1).start()
        gather(b, rows0, sem0).wait()
        s0 = reduce_one(rows0)
        gather(b + 2, rows0, sem0).start()
        gather(b + 1, rows1, sem1).wait()
        s1 = reduce_one(rows1)
        j = lax.rem(i, 8)
        vec = pack(vec, 2 * j, s0)
        vec = pack(vec, 2 * j + 1, s1)

        @pl.when(j == 7)
        def _store():
            out_v[pl.ds((i // 8) * 16, 16)] = vec

        return vec

    vec = lax.fori_loop(0, BPW // 2 - 1, loop_body,
                        jnp.zeros((16,), jnp.float32))

    gather(BPW - 1, rows1, sem1).start()
    gather(BPW - 2, rows0, sem0).wait()
    s0 = reduce_one(rows0)
    gather(BPW - 1, rows1, sem1).wait()
    s1 = reduce_one(rows1)
    vec = pack(vec, 14, s0)
    vec = pack(vec, 15, s1)
    out_v[pl.ds(BPW - 16, 16)] = vec

    pltpu.sync_copy(out_v, out_hbm.at[pl.ds(wid * BPW, BPW)])


_mesh = plsc.VectorSubcoreMesh(
    core_axis_name="c", subcore_axis_name="s", num_cores=NC, num_subcores=NS)

_nn_kernel = functools.partial(
    pl.kernel,
    out_type=jax.ShapeDtypeStruct((B,), jnp.float32),
    mesh=_mesh,
    compiler_params=pltpu.CompilerParams(use_tc_tiling_on_sc=False),
    scratch_types=[
        pltpu.VMEM((BPW, L_SEQ), jnp.int32),
        pltpu.VMEM((L_SEQ, D), jnp.float32),
        pltpu.VMEM((L_SEQ, D), jnp.float32),
        pltpu.VMEM((48,), jnp.float32),
        pltpu.VMEM((BPW,), jnp.float32),
        pltpu.SemaphoreType.DMA,
        pltpu.SemaphoreType.DMA,
    ],
)(_nn_body)


@jax.jit
def kernel(x, emb, W, b):
    wb = jnp.concatenate([W.reshape(D), jnp.broadcast_to(b, (16,))])
    out = _nn_kernel(x.astype(jnp.int32), emb, wb)
    return out.reshape(B, 1)


# speedup vs baseline: 7.4821x; 7.3101x over previous
import functools

import jax
import jax.numpy as jnp
from jax import lax
from jax.experimental import pallas as pl
from jax.experimental.pallas import tpu as pltpu
from jax.experimental.pallas import tpu_sc as plsc

VOCAB = 1000000
D = 32
B = 4096
L_SEQ = 50
NC = 2
NS = 16
NW = NC * NS
BPW = B // NW


VBLK = 32768
VGRID = -(-VOCAB // VBLK)


def _matvec_body(w_ref, embt_ref, p_ref):
    p_ref[...] = jnp.dot(w_ref[...], embt_ref[...],
                         preferred_element_type=jnp.float32)[0]


_matvec = pl.pallas_call(
    _matvec_body,
    grid=(VGRID,),
    in_specs=[
        pl.BlockSpec((1, D), lambda i: (0, 0)),
        pl.BlockSpec((D, VBLK), lambda i: (0, i)),
    ],
    out_specs=pl.BlockSpec((VBLK,), lambda i: (i,)),
    out_shape=jax.ShapeDtypeStruct((VGRID * VBLK,), jnp.float32),
)


def _pool_body(xt_hbm, p_hbm, wb_hbm, out_hbm, idx_v, val_v, wb_v, out_v, sem):
    wid = lax.axis_index("s") * NC + lax.axis_index("c")
    base = wid * BPW

    pltpu.sync_copy(xt_hbm.at[:, pl.ds(base, BPW)], idx_v)
    pltpu.sync_copy(wb_hbm, wb_v)
    bias = wb_v[pl.ds(0, 16)]
    inv_l = jnp.float32(1.0 / L_SEQ)

    for l in range(L_SEQ):
        pltpu.make_async_copy(p_hbm.at[idx_v.at[l]], val_v.at[l], sem).start()
    for l in range(L_SEQ):
        pltpu.make_async_copy(p_hbm.at[idx_v.at[l]], val_v.at[l], sem).wait()

    for j in range(BPW // 16):
        acc = val_v[0, pl.ds(j * 16, 16)]
        for l in range(1, L_SEQ):
            acc = acc + val_v[l, pl.ds(j * 16, 16)]
        out_v[pl.ds(j * 16, 16)] = acc * inv_l + bias

    pltpu.sync_copy(out_v, out_hbm.at[pl.ds(base, BPW)])


_mesh = plsc.VectorSubcoreMesh(
    core_axis_name="c", subcore_axis_name="s", num_cores=NC, num_subcores=NS)

_pool = functools.partial(
    pl.kernel,
    out_type=jax.ShapeDtypeStruct((B,), jnp.float32),
    mesh=_mesh,
    compiler_params=pltpu.CompilerParams(use_tc_tiling_on_sc=False),
    scratch_types=[
        pltpu.VMEM((L_SEQ, BPW), jnp.int32),
        pltpu.VMEM((L_SEQ, BPW), jnp.float32),
        pltpu.VMEM((16,), jnp.float32),
        pltpu.VMEM((BPW,), jnp.float32),
        pltpu.SemaphoreType.DMA,
    ],
)(_pool_body)


@jax.jit
def kernel(x, emb, W, b):
    p = _matvec(W, emb.T)
    wb = jnp.broadcast_to(b, (16,))
    out = _pool(x.astype(jnp.int32).T, p, wb)
    return out.reshape(B, 1)


# speedup vs baseline: 7.9008x; 1.0560x over previous
import functools

import jax
import jax.numpy as jnp
from jax import lax
from jax.experimental import pallas as pl
from jax.experimental.pallas import tpu as pltpu
from jax.experimental.pallas import tpu_sc as plsc

VOCAB = 1000000
D = 32
B = 4096
L_SEQ = 50
NC = 2
NS = 16
NW = NC * NS
BPW = B // NW


VBLK = 65536
VGRID = -(-VOCAB // VBLK)


def _matvec_body(w_ref, embt_ref, p_ref):
    p_ref[...] = jnp.dot(w_ref[...], embt_ref[...],
                         preferred_element_type=jnp.float32)[0]


_matvec = pl.pallas_call(
    _matvec_body,
    grid=(VGRID,),
    in_specs=[
        pl.BlockSpec((1, D), lambda i: (0, 0)),
        pl.BlockSpec((D, VBLK), lambda i: (0, i)),
    ],
    out_specs=pl.BlockSpec((VBLK,), lambda i: (i,)),
    out_shape=jax.ShapeDtypeStruct((VGRID * VBLK,), jnp.float32),
)


def _pool_body(xt_hbm, p_hbm, wb_hbm, out_hbm, idx_v, val_v, wb_v, out_v, sem):
    wid = lax.axis_index("s") * NC + lax.axis_index("c")
    base = wid * BPW

    pltpu.sync_copy(xt_hbm.at[:, pl.ds(base, BPW)], idx_v)
    pltpu.sync_copy(wb_hbm, wb_v)
    bias = wb_v[pl.ds(0, 16)]
    inv_l = jnp.float32(1.0 / L_SEQ)

    for l in range(L_SEQ):
        pltpu.make_async_copy(p_hbm.at[idx_v.at[l]], val_v.at[l], sem).start()
    for l in range(L_SEQ):
        pltpu.make_async_copy(p_hbm.at[idx_v.at[l]], val_v.at[l], sem).wait()

    for j in range(BPW // 16):
        acc = val_v[0, pl.ds(j * 16, 16)]
        for l in range(1, L_SEQ):
            acc = acc + val_v[l, pl.ds(j * 16, 16)]
        out_v[pl.ds(j * 16, 16)] = acc * inv_l + bias

    pltpu.sync_copy(out_v, out_hbm.at[pl.ds(base, BPW)])


_mesh = plsc.VectorSubcoreMesh(
    core_axis_name="c", subcore_axis_name="s", num_cores=NC, num_subcores=NS)

_pool = functools.partial(
    pl.kernel,
    out_type=jax.ShapeDtypeStruct((B,), jnp.float32),
    mesh=_mesh,
    compiler_params=pltpu.CompilerParams(use_tc_tiling_on_sc=False),
    scratch_types=[
        pltpu.VMEM((L_SEQ, BPW), jnp.int32),
        pltpu.VMEM((L_SEQ, BPW), jnp.float32),
        pltpu.VMEM((16,), jnp.float32),
        pltpu.VMEM((BPW,), jnp.float32),
        pltpu.SemaphoreType.DMA,
    ],
)(_pool_body)


@jax.jit
def kernel(x, emb, W, b):
    p = _matvec(W, emb.T)
    wb = jnp.broadcast_to(b, (16,))
    out = _pool(x.astype(jnp.int32).T, p, wb)
    return out.reshape(B, 1)
